# Initial kernel scaffold; baseline (speedup 1.0000x reference)
import jax
import jax.numpy as jnp
from jax import lax
from jax.experimental import pallas as pl
from jax.experimental.pallas import tpu as pltpu
from jax.experimental.pallas import tpu_sc as plsc

BATCH = 4096
SEQ = 100
DIM = 128
NC = 2
NS = 16
L = 16
NW = NC * NS
ROWS_PER_TILE = BATCH // NW
R = 4
NCHUNKS = ROWS_PER_TILE // R


def _body(ids_hbm, table_hbm, pos_hbm, cls_hbm, out_hbm,
          pos_v, ids_v, obuf, sem):
    wid = lax.axis_index("s") * NC + lax.axis_index("c")
    tile_base = wid * ROWS_PER_TILE

    pltpu.sync_copy(pos_hbm, pos_v)
    for r in range(R):
        pltpu.sync_copy(cls_hbm, obuf.at[r, pl.ds(0, 1)])

    @pl.loop(0, NCHUNKS)
    def chunk(i):
        base = tile_base + i * R
        pltpu.sync_copy(ids_hbm.at[pl.ds(base, R)], ids_v)
        for r in range(R):
            pltpu.async_copy(
                table_hbm.at[ids_v.at[r]],
                obuf.at[r, pl.ds(1, SEQ)],
                sem,
            ).wait()

        @pl.loop(0, SEQ)
        def pos_loop(s):
            for j in range(DIM // L):
                p = pos_v[s, pl.ds(j * L, L)]
                for r in range(R):
                    v = obuf[r, s + 1, pl.ds(j * L, L)]
                    obuf[r, s + 1, pl.ds(j * L, L)] = v + p

        pltpu.sync_copy(obuf, out_hbm.at[pl.ds(base, R)])


def kernel(bin_ids, bin_table, pos_table, cls_table):
    mesh = plsc.VectorSubcoreMesh(core_axis_name="c", subcore_axis_name="s")
    f = pl.kernel(
        _body,
        out_type=jax.ShapeDtypeStruct((BATCH, SEQ + 1, DIM), jnp.float32),
        mesh=mesh,
        scratch_types=[
            pltpu.VMEM((SEQ, DIM), jnp.float32),
            pltpu.VMEM((R, SEQ), jnp.int32),
            pltpu.VMEM((R, SEQ + 1, DIM), jnp.float32),
            pltpu.SemaphoreType.DMA,
        ],
    )
    return f(bin_ids, bin_table, pos_table, cls_table)


# speedup vs baseline: 7.4763x; 7.4763x over previous
import jax
import jax.numpy as jnp
from jax import lax
from jax.experimental import pallas as pl
from jax.experimental.pallas import tpu as pltpu
from jax.experimental.pallas import tpu_sc as plsc

BATCH = 4096
SEQ = 100
DIM = 128
NC = 2
NS = 16
L = 16
NW = NC * NS
ROWS_PER_TILE = BATCH // NW
R = 4
NCHUNKS = ROWS_PER_TILE // R


def _body(ids_hbm, table_hbm, pos_hbm, cls_hbm, out_hbm,
          pos_v, ids_v, obuf, sem):
    wid = lax.axis_index("s") * NC + lax.axis_index("c")
    tile_base = wid * ROWS_PER_TILE

    pltpu.sync_copy(pos_hbm, pos_v)
    for r in range(R):
        pltpu.sync_copy(cls_hbm, obuf.at[r, pl.ds(0, 1)])

    @pl.loop(0, NCHUNKS)
    def chunk(i):
        base = tile_base + i * R
        pltpu.sync_copy(ids_hbm.at[pl.ds(base, R)], ids_v)
        for r in range(R):
            pltpu.async_copy(
                table_hbm.at[ids_v.at[r]],
                obuf.at[r, pl.ds(1, SEQ)],
                sem,
            ).wait()

        @pl.loop(0, SEQ)
        def pos_loop(s):
            for j in range(DIM // L):
                p = pos_v[s, pl.ds(j * L, L)]
                for r in range(R):
                    v = obuf[r, s + 1, pl.ds(j * L, L)]
                    obuf[r, s + 1, pl.ds(j * L, L)] = v + p

        pltpu.sync_copy(obuf, out_hbm.at[pl.ds(base, R)])


def kernel(bin_ids, bin_table, pos_table, cls_table):
    mesh = plsc.VectorSubcoreMesh(
        core_axis_name="c", subcore_axis_name="s",
        num_cores=NC, num_subcores=NS,
    )
    f = pl.kernel(
        _body,
        out_type=jax.ShapeDtypeStruct((BATCH, SEQ + 1, DIM), jnp.float32),
        mesh=mesh,
        scratch_types=[
            pltpu.VMEM((SEQ, DIM), jnp.float32),
            pltpu.VMEM((R, SEQ), jnp.int32),
            pltpu.VMEM((R, SEQ + 1, DIM), jnp.float32),
            pltpu.SemaphoreType.DMA,
        ],
    )
    return f(bin_ids, bin_table, pos_table, cls_table)


# speedup vs baseline: 8.5289x; 1.1408x over previous
import jax
import jax.numpy as jnp
from jax import lax
from jax.experimental import pallas as pl
from jax.experimental.pallas import tpu as pltpu
from jax.experimental.pallas import tpu_sc as plsc

BATCH = 4096
SEQ = 100
DIM = 128
NC = 2
NS = 16
L = 16
NW = NC * NS
ROWS_PER_TILE = BATCH // NW
R = 2
NCHUNKS = ROWS_PER_TILE // R
NBUF = 4
NSTEPS = NCHUNKS // NBUF


def _body(ids_hbm, table_hbm, pos_hbm, cls_hbm, out_hbm,
          pos_v, ids_v, obuf, gsems, osems):
    wid = lax.axis_index("s") * NC + lax.axis_index("c")
    tile_base = wid * ROWS_PER_TILE

    pltpu.sync_copy(pos_hbm, pos_v)
    for b in range(NBUF):
        for r in range(R):
            pltpu.sync_copy(cls_hbm, obuf.at[b, r, pl.ds(0, 1)])

    def fire_g(c, b):
        pltpu.sync_copy(ids_hbm.at[pl.ds(tile_base + c * R, R)], ids_v.at[b])
        for r in range(R):
            pltpu.async_copy(
                table_hbm.at[ids_v.at[b, r]],
                obuf.at[b, r, pl.ds(1, SEQ)],
                gsems[b],
            )

    def wait_g(b):
        for r in range(R):
            pltpu.make_async_copy(
                table_hbm.at[ids_v.at[b, r]],
                obuf.at[b, r, pl.ds(1, SEQ)],
                gsems[b],
            ).wait()

    def fire_out(c, b):
        pltpu.async_copy(
            obuf.at[b], out_hbm.at[pl.ds(tile_base + c * R, R)], osems[b])

    def wait_out(c, b):
        pltpu.make_async_copy(
            obuf.at[b], out_hbm.at[pl.ds(tile_base + c * R, R)], osems[b]
        ).wait()

    def compute(b):
        @pl.loop(0, SEQ)
        def pos_loop(s):
            for j in range(DIM // L):
                p = pos_v[s, pl.ds(j * L, L)]
                for r in range(R):
                    v = obuf[b, r, s + 1, pl.ds(j * L, L)]
                    obuf[b, r, s + 1, pl.ds(j * L, L)] = v + p

    def step(c, b, do_wait_out, do_fire_g):
        if do_wait_out:
            wait_out(c - 2, (b + 2) % NBUF)
        if do_fire_g:
            fire_g(c + 2, (b + 2) % NBUF)
        wait_g(b)
        compute(b)
        fire_out(c, b)

    def superstep(i, first, last):
        for b in range(NBUF):
            c = i + b
            step(c, b,
                 do_wait_out=(not first) or b >= 2,
                 do_fire_g=(not last) or b < 2)

    fire_g(0, 0)
    fire_g(1, 1)
    superstep(0, first=True, last=False)

    @pl.loop(NBUF, (NSTEPS - 1) * NBUF, step=NBUF)
    def main(i):
        superstep(i, first=False, last=False)

    superstep((NSTEPS - 1) * NBUF, first=False, last=True)
    wait_out(NCHUNKS - 2, (NCHUNKS - 2) % NBUF)
    wait_out(NCHUNKS - 1, (NCHUNKS - 1) % NBUF)


def kernel(bin_ids, bin_table, pos_table, cls_table):
    mesh = plsc.VectorSubcoreMesh(
        core_axis_name="c", subcore_axis_name="s",
        num_cores=NC, num_subcores=NS,
    )
    f = pl.kernel(
        _body,
        out_type=jax.ShapeDtypeStruct((BATCH, SEQ + 1, DIM), jnp.float32),
        mesh=mesh,
        scratch_types=[
            pltpu.VMEM((SEQ, DIM), jnp.float32),
            pltpu.VMEM((NBUF, R, SEQ), jnp.int32),
            pltpu.VMEM((NBUF, R, SEQ + 1, DIM), jnp.float32),
            [pltpu.SemaphoreType.DMA] * NBUF,
            [pltpu.SemaphoreType.DMA] * NBUF,
        ],
    )
    return f(bin_ids, bin_table, pos_table, cls_table)


# speedup vs baseline: 8.5323x; 1.0004x over previous
import jax
import jax.numpy as jnp
from jax import lax
from jax.experimental import pallas as pl
from jax.experimental.pallas import tpu as pltpu
from jax.experimental.pallas import tpu_sc as plsc

BATCH = 4096
SEQ = 100
DIM = 128
NC = 2
NS = 16
L = 16
NW = NC * NS
ROWS_PER_TILE = BATCH // NW
R = 2
NCHUNKS = ROWS_PER_TILE // R
NBUF = 4
NSTEPS = NCHUNKS // NBUF


def _body(ids_hbm, table_hbm, pos_hbm, cls_hbm, out_hbm,
          pos_v, ids_v, obuf, gsems, osems):
    wid = lax.axis_index("s") * NC + lax.axis_index("c")
    tile_base = wid * ROWS_PER_TILE

    pltpu.sync_copy(pos_hbm, pos_v)
    for b in range(NBUF):
        for r in range(R):
            pltpu.sync_copy(cls_hbm, obuf.at[b, r, pl.ds(0, 1)])

    def fire_g(c, b):
        pltpu.sync_copy(ids_hbm.at[pl.ds(tile_base + c * R, R)], ids_v.at[b])
        for r in range(R):
            pltpu.async_copy(
                table_hbm.at[ids_v.at[b, r]],
                obuf.at[b, r, pl.ds(1, SEQ)],
                gsems[b],
            )

    def wait_g(b):
        for r in range(R):
            pltpu.make_async_copy(
                table_hbm.at[ids_v.at[b, r]],
                obuf.at[b, r, pl.ds(1, SEQ)],
                gsems[b],
            ).wait()

    def fire_out(c, b):
        pltpu.async_copy(
            obuf.at[b], out_hbm.at[pl.ds(tile_base + c * R, R)], osems[b])

    def wait_out(c, b):
        pltpu.make_async_copy(
            obuf.at[b], out_hbm.at[pl.ds(tile_base + c * R, R)], osems[b]
        ).wait()

    def compute(b):
        @pl.loop(0, SEQ)
        def pos_loop(s):
            for j in range(DIM // L):
                p = pos_v[s, pl.ds(j * L, L)]
                for r in range(R):
                    v = obuf[b, r, s + 1, pl.ds(j * L, L)]
                    obuf[b, r, s + 1, pl.ds(j * L, L)] = v + p

    def step(c, b, do_wait_out, do_fire_g):
        if do_wait_out:
            wait_out(c - 2, (b + 2) % NBUF)
        if do_fire_g:
            fire_g(c + 2, (b + 2) % NBUF)
        wait_g(b)
        compute(b)
        fire_out(c, b)

    def superstep(i, first, last):
        for b in range(NBUF):
            c = i + b
            step(c, b,
                 do_wait_out=(not first) or b >= 2,
                 do_fire_g=(not last) or b < 2)

    fire_g(0, 0)
    fire_g(1, 1)
    superstep(0, first=True, last=False)

    @pl.loop(NBUF, (NSTEPS - 1) * NBUF, step=NBUF)
    def main(i):
        superstep(i, first=False, last=False)

    superstep((NSTEPS - 1) * NBUF, first=False, last=True)
    wait_out(NCHUNKS - 2, (NCHUNKS - 2) % NBUF)
    wait_out(NCHUNKS - 1, (NCHUNKS - 1) % NBUF)


def kernel(bin_ids, bin_table, pos_table, cls_table):
    mesh = plsc.VectorSubcoreMesh(
        core_axis_name="c", subcore_axis_name="s",
        num_cores=NC, num_subcores=NS,
    )
    f = pl.kernel(
        _body,
        out_type=jax.ShapeDtypeStruct((BATCH, SEQ + 1, DIM), jnp.float32),
        mesh=mesh,
        compiler_params=pltpu.CompilerParams(use_tc_tiling_on_sc=True),
        scratch_types=[
            pltpu.VMEM((SEQ, DIM), jnp.float32),
            pltpu.VMEM((NBUF, R, SEQ), jnp.int32),
            pltpu.VMEM((NBUF, R, SEQ + 1, DIM), jnp.float32),
            [pltpu.SemaphoreType.DMA] * NBUF,
            [pltpu.SemaphoreType.DMA] * NBUF,
        ],
    )
    return f(bin_ids, bin_table, pos_table, cls_table)


# speedup vs baseline: 13.1301x; 1.5389x over previous
import jax
import jax.numpy as jnp
from jax import lax
from jax.experimental import pallas as pl
from jax.experimental.pallas import tpu as pltpu
from jax.experimental.pallas import tpu_sc as plsc

BATCH = 4096
SEQ = 100
DIM = 128
NC = 2
NS = 16
L = 16
NW = NC * NS
CHUNK = BATCH // NW
NBUF = 4


def _body(ids_hbm, table_hbm, pos_hbm, cls_hbm, out_hbm,
          pos_v, ids_v, obuf, cbuf, gsems, osems, csem):
    wid = lax.axis_index("s") * NC + lax.axis_index("c")
    cbase = wid * CHUNK

    pltpu.sync_copy(pos_hbm, pos_v)

    pltpu.sync_copy(cls_hbm, cbuf.at[pl.ds(0, 1)])
    cls_regs = [cbuf[0, pl.ds(j * L, L)] for j in range(DIM // L)]

    @pl.loop(1, CHUNK)
    def cls_fill(r):
        for j in range(DIM // L):
            cbuf[r, pl.ds(j * L, L)] = cls_regs[j]

    pltpu.async_copy(cbuf, out_hbm.at[0, pl.ds(cbase, CHUNK)], csem)

    def fire_g(c, b):
        pltpu.sync_copy(ids_hbm.at[c - 1, pl.ds(cbase, CHUNK)], ids_v.at[b])
        pltpu.async_copy(table_hbm.at[ids_v.at[b]], obuf.at[b], gsems[b])

    def wait_g(b):
        pltpu.make_async_copy(
            table_hbm.at[ids_v.at[b]], obuf.at[b], gsems[b]).wait()

    def fire_out(c, b):
        pltpu.async_copy(
            obuf.at[b], out_hbm.at[c, pl.ds(cbase, CHUNK)], osems[b])

    def wait_out(c, b):
        pltpu.make_async_copy(
            obuf.at[b], out_hbm.at[c, pl.ds(cbase, CHUNK)], osems[b]).wait()

    def compute(c, b):
        p = [pos_v[c - 1, pl.ds(j * L, L)] for j in range(DIM // L)]

        @pl.loop(0, CHUNK, unroll=4)
        def add_pos(r):
            for j in range(DIM // L):
                v = obuf[b, r, pl.ds(j * L, L)]
                obuf[b, r, pl.ds(j * L, L)] = v + p[j]

    def step(c, b, do_wait_out, do_fire_g):
        if do_wait_out:
            wait_out(c - 2, (b + 2) % NBUF)
        if do_fire_g:
            fire_g(c + 2, (b + 2) % NBUF)
        wait_g(b)
        compute(c, b)
        fire_out(c, b)

    fire_g(1, 0)
    fire_g(2, 1)
    step(1, 0, do_wait_out=False, do_fire_g=True)
    step(2, 1, do_wait_out=False, do_fire_g=True)

    @pl.loop(3, SEQ - 1, step=NBUF)
    def main(i):
        for k in range(NBUF):
            c = i + k
            step(c, (k + 2) % NBUF, do_wait_out=True, do_fire_g=True)

    step(SEQ - 1, (SEQ - 2) % NBUF, do_wait_out=True, do_fire_g=False)
    step(SEQ, (SEQ - 1) % NBUF, do_wait_out=True, do_fire_g=False)
    wait_out(SEQ - 1, (SEQ - 2) % NBUF)
    wait_out(SEQ, (SEQ - 1) % NBUF)
    pltpu.make_async_copy(cbuf, out_hbm.at[0, pl.ds(cbase, CHUNK)], csem).wait()


def kernel(bin_ids, bin_table, pos_table, cls_table):
    ids_t = jnp.transpose(bin_ids)
    mesh = plsc.VectorSubcoreMesh(
        core_axis_name="c", subcore_axis_name="s",
        num_cores=NC, num_subcores=NS,
    )
    f = pl.kernel(
        _body,
        out_type=jax.ShapeDtypeStruct((SEQ + 1, BATCH, DIM), jnp.float32),
        mesh=mesh,
        scratch_types=[
            pltpu.VMEM((SEQ, DIM), jnp.float32),
            pltpu.VMEM((NBUF, CHUNK), jnp.int32),
            pltpu.VMEM((NBUF, CHUNK, DIM), jnp.float32),
            pltpu.VMEM((CHUNK, DIM), jnp.float32),
            [pltpu.SemaphoreType.DMA] * NBUF,
            [pltpu.SemaphoreType.DMA] * NBUF,
            pltpu.SemaphoreType.DMA,
        ],
    )
    out_t = f(ids_t, bin_table, pos_table, cls_table)
    return jnp.transpose(out_t, (1, 0, 2))


# speedup vs baseline: 24.2864x; 1.8497x over previous
import jax
import jax.numpy as jnp
from jax import lax
from jax.experimental import pallas as pl
from jax.experimental.pallas import tpu as pltpu
from jax.experimental.pallas import tpu_sc as plsc

BATCH = 4096
SEQ = 100
DIM = 128
NC = 2
NS = 16
L = 16
NW = NC * NS
CHUNK = BATCH // NW
NBUF = 4


def _body(ids_hbm, table_hbm, pos_hbm, cls_hbm, out_hbm,
          pos_v, ids_v, obuf, cbuf, table_s, gsems, osems, csem):
    sid = lax.axis_index("s")
    wid = sid * NC + lax.axis_index("c")
    cbase = wid * CHUNK

    @pl.when(sid == 0)
    def _stage():
        pltpu.sync_copy(table_hbm, table_s)

    plsc.subcore_barrier()

    pltpu.sync_copy(pos_hbm, pos_v)

    pltpu.sync_copy(cls_hbm, cbuf.at[pl.ds(0, 1)])
    cls_regs = [cbuf[0, pl.ds(j * L, L)] for j in range(DIM // L)]

    @pl.loop(1, CHUNK)
    def cls_fill(r):
        for j in range(DIM // L):
            cbuf[r, pl.ds(j * L, L)] = cls_regs[j]

    pltpu.async_copy(cbuf, out_hbm.at[0, pl.ds(cbase, CHUNK)], csem)

    def fire_g(c, b):
        pltpu.sync_copy(ids_hbm.at[c - 1, pl.ds(cbase, CHUNK)], ids_v.at[b])
        pltpu.async_copy(table_s.at[ids_v.at[b]], obuf.at[b], gsems[b])

    def wait_g(b):
        pltpu.make_async_copy(
            table_s.at[ids_v.at[b]], obuf.at[b], gsems[b]).wait()

    def fire_out(c, b):
        pltpu.async_copy(
            obuf.at[b], out_hbm.at[c, pl.ds(cbase, CHUNK)], osems[b])

    def wait_out(c, b):
        pltpu.make_async_copy(
            obuf.at[b], out_hbm.at[c, pl.ds(cbase, CHUNK)], osems[b]).wait()

    def compute(c, b):
        p = [pos_v[c - 1, pl.ds(j * L, L)] for j in range(DIM // L)]

        @pl.loop(0, CHUNK, unroll=4)
        def add_pos(r):
            for j in range(DIM // L):
                v = obuf[b, r, pl.ds(j * L, L)]
                obuf[b, r, pl.ds(j * L, L)] = v + p[j]

    def step(c, b, do_wait_out, do_fire_g):
        if do_wait_out:
            wait_out(c - 2, (b + 2) % NBUF)
        if do_fire_g:
            fire_g(c + 2, (b + 2) % NBUF)
        wait_g(b)
        compute(c, b)
        fire_out(c, b)

    fire_g(1, 0)
    fire_g(2, 1)
    step(1, 0, do_wait_out=False, do_fire_g=True)
    step(2, 1, do_wait_out=False, do_fire_g=True)

    @pl.loop(3, SEQ - 1, step=NBUF)
    def main(i):
        for k in range(NBUF):
            c = i + k
            step(c, (k + 2) % NBUF, do_wait_out=True, do_fire_g=True)

    step(SEQ - 1, (SEQ - 2) % NBUF, do_wait_out=True, do_fire_g=False)
    step(SEQ, (SEQ - 1) % NBUF, do_wait_out=True, do_fire_g=False)
    wait_out(SEQ - 1, (SEQ - 2) % NBUF)
    wait_out(SEQ, (SEQ - 1) % NBUF)
    pltpu.make_async_copy(cbuf, out_hbm.at[0, pl.ds(cbase, CHUNK)], csem).wait()


def kernel(bin_ids, bin_table, pos_table, cls_table):
    ids_t = jnp.transpose(bin_ids)
    mesh = plsc.VectorSubcoreMesh(
        core_axis_name="c", subcore_axis_name="s",
        num_cores=NC, num_subcores=NS,
    )
    f = pl.kernel(
        _body,
        out_type=jax.ShapeDtypeStruct((SEQ + 1, BATCH, DIM), jnp.float32),
        mesh=mesh,
        scratch_types=[
            pltpu.VMEM((SEQ, DIM), jnp.float32),
            pltpu.VMEM((NBUF, CHUNK), jnp.int32),
            pltpu.VMEM((NBUF, CHUNK, DIM), jnp.float32),
            pltpu.VMEM((CHUNK, DIM), jnp.float32),
            pltpu.VMEM_SHARED((1000, DIM), jnp.float32),

            [pltpu.SemaphoreType.DMA] * NBUF,
            [pltpu.SemaphoreType.DMA] * NBUF,
            pltpu.SemaphoreType.DMA,
        ],
    )
    out_t = f(ids_t, bin_table, pos_table, cls_table)
    return jnp.transpose(out_t, (1, 0, 2))


# speedup vs baseline: 26.3976x; 1.0869x over previous
import jax
import jax.numpy as jnp
from jax import lax
from jax.experimental import pallas as pl
from jax.experimental.pallas import tpu as pltpu
from jax.experimental.pallas import tpu_sc as plsc

BATCH = 4096
SEQ = 100
DIM = 128
NC = 2
NS = 16
L = 16
NW = NC * NS
CHUNK = BATCH // NW
NBUF = 4


def _body(ids_hbm, table_hbm, pos_hbm, cls_hbm, out_hbm,
          pos_v, ids_all, obuf, cbuf, table_s, gsems, osems, csem):
    sid = lax.axis_index("s")
    wid = sid * NC + lax.axis_index("c")
    cbase = wid * CHUNK

    @pl.when(sid == 0)
    def _stage():
        pltpu.sync_copy(table_hbm, table_s)

    plsc.subcore_barrier()

    pltpu.sync_copy(pos_hbm, pos_v)
    pltpu.sync_copy(ids_hbm.at[:, pl.ds(cbase, CHUNK)], ids_all)

    pltpu.sync_copy(cls_hbm, cbuf.at[pl.ds(0, 1)])
    cls_regs = [cbuf[0, pl.ds(j * L, L)] for j in range(DIM // L)]

    @pl.loop(1, CHUNK)
    def cls_fill(r):
        for j in range(DIM // L):
            cbuf[r, pl.ds(j * L, L)] = cls_regs[j]

    pltpu.async_copy(cbuf, out_hbm.at[0, pl.ds(cbase, CHUNK)], csem)

    def fire_g(c, b):
        pltpu.async_copy(table_s.at[ids_all.at[c - 1]], obuf.at[b], gsems[b])

    def wait_g(c, b):
        pltpu.make_async_copy(
            table_s.at[ids_all.at[c - 1]], obuf.at[b], gsems[b]).wait()

    def fire_out(c, b):
        pltpu.async_copy(
            obuf.at[b], out_hbm.at[c, pl.ds(cbase, CHUNK)], osems[b])

    def wait_out(c, b):
        pltpu.make_async_copy(
            obuf.at[b], out_hbm.at[c, pl.ds(cbase, CHUNK)], osems[b]).wait()

    def compute(c, b):
        p = [pos_v[c - 1, pl.ds(j * L, L)] for j in range(DIM // L)]

        @pl.loop(0, CHUNK, unroll=8)
        def add_pos(r):
            for j in range(DIM // L):
                v = obuf[b, r, pl.ds(j * L, L)]
                obuf[b, r, pl.ds(j * L, L)] = v + p[j]

    def step(c, b, do_wait_out, do_fire_g):
        if do_wait_out:
            wait_out(c - 2, (b + 2) % NBUF)
        if do_fire_g:
            fire_g(c + 2, (b + 2) % NBUF)
        wait_g(c, b)
        compute(c, b)
        fire_out(c, b)

    fire_g(1, 0)
    fire_g(2, 1)
    step(1, 0, do_wait_out=False, do_fire_g=True)
    step(2, 1, do_wait_out=False, do_fire_g=True)

    @pl.loop(3, SEQ - 1, step=NBUF)
    def main(i):
        for k in range(NBUF):
            c = i + k
            step(c, (k + 2) % NBUF, do_wait_out=True, do_fire_g=True)

    step(SEQ - 1, (SEQ - 2) % NBUF, do_wait_out=True, do_fire_g=False)
    step(SEQ, (SEQ - 1) % NBUF, do_wait_out=True, do_fire_g=False)
    wait_out(SEQ - 1, (SEQ - 2) % NBUF)
    wait_out(SEQ, (SEQ - 1) % NBUF)
    pltpu.make_async_copy(cbuf, out_hbm.at[0, pl.ds(cbase, CHUNK)], csem).wait()


def kernel(bin_ids, bin_table, pos_table, cls_table):
    ids_t = jnp.transpose(bin_ids)
    mesh = plsc.VectorSubcoreMesh(
        core_axis_name="c", subcore_axis_name="s",
        num_cores=NC, num_subcores=NS,
    )
    f = pl.kernel(
        _body,
        out_type=jax.ShapeDtypeStruct((SEQ + 1, BATCH, DIM), jnp.float32),
        mesh=mesh,
        scratch_types=[
            pltpu.VMEM((SEQ, DIM), jnp.float32),
            pltpu.VMEM((SEQ, CHUNK), jnp.int32),
            pltpu.VMEM((NBUF, CHUNK, DIM), jnp.float32),
            pltpu.VMEM((CHUNK, DIM), jnp.float32),
            pltpu.VMEM_SHARED((1000, DIM), jnp.float32),

            [pltpu.SemaphoreType.DMA] * NBUF,
            [pltpu.SemaphoreType.DMA] * NBUF,
            pltpu.SemaphoreType.DMA,
        ],
    )
    out_t = f(ids_t, bin_table, pos_table, cls_table)
    return jnp.transpose(out_t, (1, 0, 2))
